# baseline (speedup 1.0000x reference)
import functools

import jax
import jax.numpy as jnp
from jax import lax
from jax.experimental import pallas as pl
from jax.experimental.pallas import tpu as pltpu
from jax.experimental.pallas import tpu_sc as plsc

N_NODES = 10000
N_EDGES = 320000
D = 128
L = 16
NC, NS = 2, 16
N_PAD = 10240
NGRP = 16
ROWS = N_PAD // NGRP
HALF = ROWS // 2
E_PAD = 327680
EROWS = E_PAD // D
CROWS = 40
CHUNK = CROWS * D
NCHM = EROWS // CROWS // 2
BLK = 2
NBLK = CHUNK // (BLK * L)
QCAP = 128
QPAD = QCAP + L
TPAD = ROWS + L


def _body(x_hbm, edge_hbm, out_hbm,
          acc, gbuf, edgec, qs2, qd2, touched, tpart, xch, tch, sem, sem2):
  c = lax.axis_index("c")
  s = lax.axis_index("s")
  member = s // 8
  pair = s % 8
  grp = c * 8 + pair
  glo = grp * ROWS
  ghi = glo + ROWS

  zeros = jnp.zeros((L,), jnp.int32)
  neg_inf = jnp.full((L,), -jnp.inf, jnp.float32)
  one_vec = jnp.ones((L,), jnp.int32)
  lane0 = lax.iota(jnp.int32, L) == 0
  trash = jnp.full((L,), ROWS + glo, jnp.int32)

  def init_acc(r, _):
    for j in range(D // L):
      acc[r, pl.ds(j * L, L)] = neg_inf
    return 0
  lax.fori_loop(0, ROWS + 1, init_acc, 0)

  def init_t(i, _):
    touched[pl.ds(i * L, L)] = zeros
    return 0
  lax.fori_loop(0, TPAD // L, init_t, 0)
  for p in range(2):
    for i in range(QCAP // L):
      qs2[p, pl.ds(i * L, L)] = zeros
    for i in range(QPAD // L):
      qd2[p, pl.ds(i * L, L)] = trash

  def process_n(p, ngroups):
    def grp_body(g, _):
      dvec = qd2[p, pl.ds(g * L, L)]
      for e in range(L):
        r = dvec[e] - glo
        eg = g * L + e
        msg = [gbuf[p, eg, pl.ds(j * L, L)] for j in range(D // L)]
        cur = [acc[r, pl.ds(j * L, L)] for j in range(D // L)]
        for j in range(D // L):
          acc[r, pl.ds(j * L, L)] = jnp.maximum(cur[j], msg[j])
      return 0
    lax.fori_loop(0, ngroups, grp_body, 0)

  def fire(p):
    pltpu.async_copy(x_hbm.at[qs2.at[p]], gbuf.at[p], sem)

  def drain(p):
    pltpu.make_async_copy(x_hbm.at[qs2.at[p]], gbuf.at[p], sem).wait()

  fire(1)

  def make_blk_body(tb):
    def blk_body(b, carry):
      pos, par, png = carry
      row = b >> 2
      colbase = (b & 3) * (BLK * L)
      vecs = []
      for k in range(BLK):
        col = pl.ds(colbase + k * L, L)
        pv = edgec[tb, row, col]
        sv = pv & 0xFFFF
        dv = pv >> 16
        m = (dv >= glo) & (dv < ghi)
        plsc.store_scatter(touched, [dv - glo], one_vec, mask=m)
        cs = plsc.cumsum(m.astype(jnp.int32))
        vecs.append((dv, sv, m, cs, cs[L - 1]))
      for k in range(BLK):
        dv, sv, m, cs, cnt = vecs[k]
        idx = pos + cs - 1
        bv = jnp.full((L,), par, jnp.int32)
        plsc.store_scatter(qs2, [bv, idx], sv, mask=m)
        plsc.store_scatter(qd2, [bv, idx], dv, mask=m)
        pos = pos + cnt

      full = pos > QCAP - BLK * L

      def rotate(par):
        prev = 1 - par
        with jax.named_scope("drain_wait"):
          drain(prev)
        with jax.named_scope("proc"):
          process_n(prev, png)
        fire(par)
        return prev
      par = lax.cond(full, rotate, lambda q: q, par)
      png = jnp.where(full, (pos + L - 1) // L, png)
      pos = jnp.where(full, 0, pos)
      return pos, par, png
    return blk_body

  def fire_chunk(t):
    roff = pl.multiple_of((2 * t + member) * CROWS, 8)
    pltpu.async_copy(edge_hbm.at[pl.ds(roff, CROWS)], edgec.at[t & 1], sem2)

  def chunk_body(t, carry):
    @pl.when(t < NCHM - 1)
    def _():
      fire_chunk(t + 1)
    with jax.named_scope("chunk_dma"):
      pltpu.make_async_copy(edge_hbm.at[pl.ds(0, CROWS)], edgec.at[0],
                            sem2).wait()
    return lax.fori_loop(0, NBLK, make_blk_body(t & 1), carry)

  fire_chunk(0)
  pos, par, png = lax.fori_loop(0, NCHM, chunk_body, (0, 0, 0))

  prev = 1 - par
  drain(prev)
  process_n(prev, png)
  pltpu.async_copy(x_hbm.at[qs2.at[par]], gbuf.at[par], sem).wait()
  process_n(par, (pos + L - 1) // L)

  other = 1 - member
  slot = grp * 2 + member
  pslot = grp * 2 + other
  base = member * HALF

  pltpu.sync_copy(acc.at[pl.ds(other * HALF, HALF)], xch.at[slot])
  pltpu.sync_copy(touched.at[pl.ds(other * HALF, HALF)],
                  tch.at[pl.ds(pl.multiple_of(slot * HALF, 8), HALF)])
  plsc.subcore_barrier()
  for mb, mrows in ((0, QCAP), (QCAP, QCAP), (2 * QCAP, HALF - 2 * QCAP)):
    pltpu.sync_copy(xch.at[pslot, pl.ds(mb, mrows)],
                    gbuf.at[0, pl.ds(0, mrows)])

    def merge_body(r, _, mb=mb):
      for j in range(D // L):
        sl = pl.ds(j * L, L)
        acc[base + mb + r, sl] = jnp.maximum(acc[base + mb + r, sl],
                                             gbuf[0, r, sl])
      return 0
    lax.fori_loop(0, mrows, merge_body, 0)

  pltpu.sync_copy(tch.at[pl.ds(pl.multiple_of(pslot * HALF, 8), HALF)],
                  tpart.at[pl.ds(0, HALF)])

  def morrow(r, _):
    @pl.when(tpart[pl.ds(r, L)][0] > 0)
    def _():
      plsc.store_scatter(touched, [jnp.full((L,), base + r, jnp.int32)],
                         one_vec, mask=lane0)
    return 0
  lax.fori_loop(0, HALF, morrow, 0)

  for wb, wrows in ((0, QCAP), (QCAP, QCAP), (2 * QCAP, HALF - 2 * QCAP)):
    pltpu.sync_copy(x_hbm.at[pl.ds(glo + base + wb, wrows)],
                    gbuf.at[0, pl.ds(0, wrows)])

    def fix_body(r, _, wb=wb):
      @pl.when(touched[pl.ds(base + wb + r, L)][0] == 0)
      def _():
        for j in range(D // L):
          sl = pl.ds(j * L, L)
          acc[base + wb + r, sl] = gbuf[0, r, sl]
      return 0
    lax.fori_loop(0, wrows, fix_body, 0)
    pltpu.sync_copy(acc.at[pl.ds(base + wb, wrows)],
                    out_hbm.at[pl.ds(glo + base + wb, wrows)])


@jax.jit
def _gcn_sc(x_pad, edges):
  mesh = plsc.VectorSubcoreMesh(core_axis_name="c", subcore_axis_name="s",
                                num_cores=NC, num_subcores=NS)
  return pl.kernel(
      _body,
      out_type=jax.ShapeDtypeStruct((N_PAD, D), jnp.float32),
      mesh=mesh,
      compiler_params=pltpu.CompilerParams(needs_layout_passes=False),
      scratch_types=[
          pltpu.VMEM((ROWS + 1, D), jnp.float32),
          pltpu.VMEM((2, QCAP, D), jnp.float32),
          pltpu.VMEM((2, CROWS, D), jnp.int32),
          pltpu.VMEM((2, QCAP), jnp.int32),
          pltpu.VMEM((2, QPAD), jnp.int32),
          pltpu.VMEM((TPAD,), jnp.int32),
          pltpu.VMEM((HALF + L,), jnp.int32),
          pltpu.HBM((2 * NGRP, HALF, D), jnp.float32),
          pltpu.HBM((2 * NGRP * HALF,), jnp.int32),
          pltpu.SemaphoreType.DMA,
          pltpu.SemaphoreType.DMA,
      ],
  )(x_pad, edges)


def kernel(x, edge_index):
  x_pad = jnp.zeros((N_PAD, D), jnp.float32).at[:N_NODES].set(x)
  packed = edge_index[0] + (edge_index[1] << 16)
  packed = jnp.concatenate([packed, packed[:E_PAD - N_EDGES]]).reshape(
      EROWS, D)
  v = _gcn_sc(x_pad, packed)
  return jnp.concatenate([x, v[:N_NODES]], axis=1)


# speedup vs baseline: 3.3981x; 3.3981x over previous
import functools

import jax
import jax.numpy as jnp
from jax import lax
from jax.experimental import pallas as pl
from jax.experimental.pallas import tpu as pltpu
from jax.experimental.pallas import tpu_sc as plsc

N_NODES = 10000
N_EDGES = 320000
D = 128
L = 16
NC, NS = 2, 16
N_PAD = 10240
NGRP = 16
ROWS = N_PAD // NGRP
HALF = ROWS // 2
E_PAD = 327680
EROWS = E_PAD // D
CROWS = 40
CHUNK = CROWS * D
NCHM = EROWS // CROWS // 2
BLK = 2
NBLK = CHUNK // (BLK * L)
QCAP = 256
W = 2 * L
D32 = D // 2
QPAD = QCAP + L
TPAD = ROWS + L


def _body(x_hbm, edge_hbm, out_hbm,
          acc, gbuf, edgec, qs2, qd2, touched, tpart, xch, tch, sem, sem2):
  c = lax.axis_index("c")
  s = lax.axis_index("s")
  member = s // 8
  pair = s % 8
  grp = c * 8 + pair
  glo = grp * ROWS
  ghi = glo + ROWS

  zeros = jnp.zeros((L,), jnp.int32)
  neg_inf = jnp.full((L,), -8323200, jnp.int32)
  one_vec = jnp.ones((L,), jnp.int32)
  lane0 = lax.iota(jnp.int32, L) == 0
  trash = jnp.full((L,), ROWS + glo, jnp.int32)

  def init_acc(r, _):
    for j in range(D32 // L):
      acc[r, pl.ds(j * L, L)] = neg_inf
    return 0
  lax.fori_loop(0, ROWS + 1, init_acc, 0)

  def init_t(i, _):
    touched[pl.ds(i * L, L)] = zeros
    return 0
  lax.fori_loop(0, TPAD // L, init_t, 0)
  for p in range(2):
    for i in range(QCAP // L):
      qs2[p, pl.ds(i * L, L)] = zeros
    for i in range(QPAD // L):
      qd2[p, pl.ds(i * L, L)] = trash

  def process_n(p, ngroups):
    def grp_body(g, _):
      dvec = qd2[p, pl.ds(g * L, L)]
      for e in range(L):
        r = dvec[e] - glo
        eg = g * L + e
        msg = [plsc.bitcast(gbuf[p, eg, pl.ds(j * L, L)], jnp.bfloat16)
               for j in range(D32 // L)]
        cur = [plsc.bitcast(acc[r, pl.ds(j * L, L)], jnp.bfloat16)
               for j in range(D32 // L)]
        for j in range(D32 // L):
          acc[r, pl.ds(j * L, L)] = plsc.bitcast(
              jnp.maximum(cur[j], msg[j]), jnp.int32)
      return 0
    lax.fori_loop(0, ngroups, grp_body, 0)

  def fire(p):
    pltpu.async_copy(x_hbm.at[qs2.at[p]], gbuf.at[p], sem)

  def drain(p):
    pltpu.make_async_copy(x_hbm.at[qs2.at[p]], gbuf.at[p], sem).wait()

  fire(1)

  def make_blk_body(tb):
    def blk_body(b, carry):
      pos, par, png = carry
      row = b >> 2
      colbase = (b & 3) * (BLK * L)
      vecs = []
      for k in range(BLK):
        col = pl.ds(colbase + k * L, L)
        pv = edgec[tb, row, col]
        sv = pv & 0xFFFF
        dv = pv >> 16
        m = (dv >= glo) & (dv < ghi)
        plsc.store_scatter(touched, [dv - glo], one_vec, mask=m)
        cs = plsc.cumsum(m.astype(jnp.int32))
        vecs.append((dv, sv, m, cs, cs[L - 1]))
      for k in range(BLK):
        dv, sv, m, cs, cnt = vecs[k]
        idx = pos + cs - 1
        bv = jnp.full((L,), par, jnp.int32)
        plsc.store_scatter(qs2, [bv, idx], sv, mask=m)
        plsc.store_scatter(qd2, [bv, idx], dv, mask=m)
        pos = pos + cnt

      full = pos > QCAP - BLK * L

      def rotate(par):
        prev = 1 - par
        with jax.named_scope("drain_wait"):
          drain(prev)
        with jax.named_scope("proc"):
          process_n(prev, png)
        fire(par)
        return prev
      par = lax.cond(full, rotate, lambda q: q, par)
      png = jnp.where(full, (pos + L - 1) // L, png)
      pos = jnp.where(full, 0, pos)
      return pos, par, png
    return blk_body

  def fire_chunk(t):
    roff = pl.multiple_of((2 * t + member) * CROWS, 8)
    pltpu.async_copy(edge_hbm.at[pl.ds(roff, CROWS)], edgec.at[t & 1], sem2)

  def chunk_body(t, carry):
    @pl.when(t < NCHM - 1)
    def _():
      fire_chunk(t + 1)
    with jax.named_scope("chunk_dma"):
      pltpu.make_async_copy(edge_hbm.at[pl.ds(0, CROWS)], edgec.at[0],
                            sem2).wait()
    return lax.fori_loop(0, NBLK, make_blk_body(t & 1), carry)

  fire_chunk(0)
  pos, par, png = lax.fori_loop(0, NCHM, chunk_body, (0, 0, 0))

  prev = 1 - par
  drain(prev)
  process_n(prev, png)
  pltpu.async_copy(x_hbm.at[qs2.at[par]], gbuf.at[par], sem).wait()
  process_n(par, (pos + L - 1) // L)

  other = 1 - member
  slot = grp * 2 + member
  pslot = grp * 2 + other
  base = member * HALF

  pltpu.sync_copy(acc.at[pl.ds(other * HALF, HALF)], xch.at[slot])
  pltpu.sync_copy(touched.at[pl.ds(other * HALF, HALF)],
                  tch.at[pl.ds(pl.multiple_of(slot * HALF, 8), HALF)])
  plsc.subcore_barrier()
  for mb, mrows in ((0, HALF // 2), (HALF // 2, HALF // 2)):
    pltpu.sync_copy(xch.at[pslot, pl.ds(mb, mrows)],
                    gbuf.at[0, pl.ds(0, mrows)])

    def merge_body(r, _, mb=mb):
      for j in range(D32 // L):
        sl = pl.ds(j * L, L)
        a = plsc.bitcast(acc[base + mb + r, sl], jnp.bfloat16)
        g = plsc.bitcast(gbuf[0, r, sl], jnp.bfloat16)
        acc[base + mb + r, sl] = plsc.bitcast(jnp.maximum(a, g), jnp.int32)
      return 0
    lax.fori_loop(0, mrows, merge_body, 0)

  pltpu.sync_copy(tch.at[pl.ds(pl.multiple_of(pslot * HALF, 8), HALF)],
                  tpart.at[pl.ds(0, HALF)])

  def morrow(r, _):
    @pl.when(tpart[pl.ds(r, L)][0] > 0)
    def _():
      plsc.store_scatter(touched, [jnp.full((L,), base + r, jnp.int32)],
                         one_vec, mask=lane0)
    return 0
  lax.fori_loop(0, HALF, morrow, 0)

  for wb, wrows in ((0, HALF // 2), (HALF // 2, HALF // 2)):
    pltpu.sync_copy(x_hbm.at[pl.ds(glo + base + wb, wrows)],
                    gbuf.at[0, pl.ds(0, wrows)])

    def fix_body(r, _, wb=wb):
      @pl.when(touched[pl.ds(base + wb + r, L)][0] == 0)
      def _():
        for j in range(D32 // L):
          sl = pl.ds(j * L, L)
          acc[base + wb + r, sl] = gbuf[0, r, sl]
      return 0
    lax.fori_loop(0, wrows, fix_body, 0)
    pltpu.sync_copy(acc.at[pl.ds(base + wb, wrows)],
                    out_hbm.at[pl.ds(glo + base + wb, wrows)])


@jax.jit
def _gcn_sc(x_pad, edges):
  mesh = plsc.VectorSubcoreMesh(core_axis_name="c", subcore_axis_name="s",
                                num_cores=NC, num_subcores=NS)
  return pl.kernel(
      _body,
      out_type=jax.ShapeDtypeStruct((N_PAD, D32), jnp.int32),
      mesh=mesh,
      compiler_params=pltpu.CompilerParams(needs_layout_passes=False,
                                           use_tc_tiling_on_sc=False),
      scratch_types=[
          pltpu.VMEM((ROWS + 1, D32), jnp.int32),
          pltpu.VMEM((2, QCAP, D32), jnp.int32),
          pltpu.VMEM((2, CROWS, D), jnp.int32),
          pltpu.VMEM((2, QCAP), jnp.int32),
          pltpu.VMEM((2, QPAD), jnp.int32),
          pltpu.VMEM((TPAD,), jnp.int32),
          pltpu.VMEM((HALF + L,), jnp.int32),
          pltpu.HBM((2 * NGRP, HALF, D32), jnp.int32),
          pltpu.HBM((2 * NGRP * HALF,), jnp.int32),
          pltpu.SemaphoreType.DMA,
          pltpu.SemaphoreType.DMA,
      ],
  )(x_pad, edges)


def kernel(x, edge_index):
  x_pad = jnp.zeros((N_PAD, D), jnp.bfloat16).at[:N_NODES].set(
      x.astype(jnp.bfloat16))
  x32 = lax.bitcast_convert_type(x_pad.reshape(N_PAD, D32, 2), jnp.int32)
  packed = edge_index[0] + (edge_index[1] << 16)
  packed = jnp.concatenate([packed, packed[:E_PAD - N_EDGES]]).reshape(
      EROWS, D)
  v32 = _gcn_sc(x32, packed)
  v = lax.bitcast_convert_type(v32, jnp.bfloat16).reshape(N_PAD, D)
  return jnp.concatenate([x, v[:N_NODES].astype(jnp.float32)], axis=1)
